# baseline (speedup 1.0000x reference)
import functools

import jax
import jax.numpy as jnp
from jax import lax
from jax.experimental import pallas as pl
from jax.experimental.pallas import tpu as pltpu
from jax.experimental.pallas import tpu_sc as plsc

B, L, D, V = 512, 512, 128, 1024
PAD_ID, EOS_ID = 0, 1
N = B * L
NC, NS = 2, 16
NW = NC * NS
CHUNK = 128
CPW = N // (NW * CHUNK)
IPW = CPW * CHUNK

_mesh = plsc.VectorSubcoreMesh(core_axis_name="c", subcore_axis_name="s")


@functools.partial(
    pl.kernel,
    out_type=(
        jax.ShapeDtypeStruct((N, D), jnp.float32),
        jax.ShapeDtypeStruct((N,), jnp.int32),
        jax.ShapeDtypeStruct((N,), jnp.float32),
    ),
    mesh=_mesh,
    scratch_types=[
        pltpu.VMEM((IPW,), jnp.int32),
        pltpu.VMEM((2, CHUNK, D), jnp.float32),
        pltpu.VMEM((IPW,), jnp.int32),
        pltpu.VMEM((IPW,), jnp.float32),
        pltpu.VMEM_SHARED((V, D), jnp.float32),
        pltpu.SemaphoreType.DMA,
        pltpu.SemaphoreType.DMA,
        pltpu.SemaphoreType.DMA,
        pltpu.SemaphoreType.DMA,
        pltpu.SemaphoreType.DMA,
        pltpu.SemaphoreType.DMA,
    ],
)
def _gather_sc(ids_hbm, table_hbm, out_hbm, mask_hbm, eos_hbm,
               idx_v, rows_v, mask_v, eos_v, tab_sh, g0, g1, o0, o1, ms, es):
    wid = lax.axis_index("s") * NC + lax.axis_index("c")
    base_id = wid * IPW

    @pl.when(lax.axis_index("s") == 0)
    def _():
        pltpu.sync_copy(table_hbm, tab_sh)

    pltpu.sync_copy(ids_hbm.at[pl.ds(base_id, IPW)], idx_v)
    plsc.subcore_barrier()

    def gather(j, slot, sem):
        return pltpu.async_copy(
            tab_sh.at[idx_v.at[pl.ds(j * CHUNK, CHUNK)]], rows_v.at[slot], sem)

    def put(j, slot, sem):
        return pltpu.async_copy(
            rows_v.at[slot],
            out_hbm.at[pl.ds(base_id + j * CHUNK, CHUNK)], sem)

    gather(0, 0, g0)
    gather(1, 1, g1)

    def me_body(t, _):
        base = t * 64
        ones_i = jnp.full((16,), 1, jnp.int32)
        zeros_i = jnp.full((16,), 0, jnp.int32)
        ones_f = jnp.full((16,), 1.0, jnp.float32)
        zeros_f = jnp.full((16,), 0.0, jnp.float32)
        for u in range(4):
            v = idx_v[pl.ds(base + 16 * u, 16)]
            mask_v[pl.ds(base + 16 * u, 16)] = jnp.where(
                v == PAD_ID, ones_i, zeros_i)
            eos_v[pl.ds(base + 16 * u, 16)] = jnp.where(
                v == EOS_ID, ones_f, zeros_f)
        return 0

    lax.fori_loop(0, IPW // 64, me_body, 0)
    pltpu.async_copy(mask_v, mask_hbm.at[pl.ds(base_id, IPW)], ms)
    pltpu.async_copy(eos_v, eos_hbm.at[pl.ds(base_id, IPW)], es)

    def body(i, _):
        j0 = 2 * i
        pltpu.make_async_copy(
            tab_sh.at[idx_v.at[pl.ds(0, CHUNK)]], rows_v.at[0], g0).wait()
        put(j0, 0, o0)
        pltpu.make_async_copy(
            tab_sh.at[idx_v.at[pl.ds(0, CHUNK)]], rows_v.at[1], g1).wait()
        put(j0 + 1, 1, o1)

        @pl.when(i + 1 < CPW // 2)
        def _():
            pltpu.make_async_copy(
                rows_v.at[0], out_hbm.at[pl.ds(0, CHUNK)], o0).wait()
            gather(j0 + 2, 0, g0)
            pltpu.make_async_copy(
                rows_v.at[1], out_hbm.at[pl.ds(0, CHUNK)], o1).wait()
            gather(j0 + 3, 1, g1)

        return 0

    lax.fori_loop(0, CPW // 2, body, 0)
    pltpu.make_async_copy(rows_v.at[0], out_hbm.at[pl.ds(0, CHUNK)], o0).wait()
    pltpu.make_async_copy(rows_v.at[1], out_hbm.at[pl.ds(0, CHUNK)], o1).wait()
    pltpu.make_async_copy(mask_v, mask_hbm.at[pl.ds(0, IPW)], ms).wait()
    pltpu.make_async_copy(eos_v, eos_hbm.at[pl.ds(0, IPW)], es).wait()


def kernel(lookup_ids, embedding_matrix):
    ids_flat = lookup_ids.reshape(N)
    rows, mask32, eos = _gather_sc(ids_flat, embedding_matrix)
    matrices = rows.reshape(B, L, D)
    mask = mask32.reshape(B, L).astype(jnp.bool_)
    eos_positions = eos.reshape(B, L)
    return (matrices, mask, eos_positions, embedding_matrix, lookup_ids)


# speedup vs baseline: 1.0865x; 1.0865x over previous
import functools

import jax
import jax.numpy as jnp
from jax import lax
from jax.experimental import pallas as pl
from jax.experimental.pallas import tpu as pltpu
from jax.experimental.pallas import tpu_sc as plsc

B, L, D, V = 512, 512, 128, 1024
PAD_ID, EOS_ID = 0, 1
N = B * L
NC, NS = 2, 16
NW = NC * NS
CHUNK = 128
GPS = 2
SLOT = GPS * CHUNK
SPW = N // (NW * SLOT)
IPW = N // NW

_mesh = plsc.VectorSubcoreMesh(core_axis_name="c", subcore_axis_name="s")


@functools.partial(
    pl.kernel,
    out_type=jax.ShapeDtypeStruct((N, D), jnp.float32),
    mesh=_mesh,
    scratch_types=[
        pltpu.VMEM((IPW,), jnp.int32),
        pltpu.VMEM((2, SLOT, D), jnp.float32),
        pltpu.VMEM_SHARED((V, D), jnp.float32),
        pltpu.SemaphoreType.DMA,
        pltpu.SemaphoreType.DMA,
        pltpu.SemaphoreType.DMA,
        pltpu.SemaphoreType.DMA,
    ],
)
def _gather_sc(ids_hbm, table_hbm, out_hbm, idx_v, rows_v, tab_sh, g0, g1, o0, o1):
    wid = lax.axis_index("s") * NC + lax.axis_index("c")
    base_id = wid * IPW

    @pl.when(lax.axis_index("s") == 0)
    def _():
        pltpu.sync_copy(table_hbm, tab_sh)

    pltpu.sync_copy(ids_hbm.at[pl.ds(base_id, IPW)], idx_v)
    plsc.subcore_barrier()

    def gather(j, slot, sem):
        for u in range(GPS):
            pltpu.async_copy(
                tab_sh.at[idx_v.at[pl.ds(j * SLOT + u * CHUNK, CHUNK)]],
                rows_v.at[slot].at[pl.ds(u * CHUNK, CHUNK)], sem)

    def wait_gather(slot, sem):
        for _ in range(GPS):
            pltpu.make_async_copy(
                tab_sh.at[idx_v.at[pl.ds(0, CHUNK)]],
                rows_v.at[slot].at[pl.ds(0, CHUNK)], sem).wait()

    def put(j, slot, sem):
        pltpu.async_copy(
            rows_v.at[slot],
            out_hbm.at[pl.ds(base_id + j * SLOT, SLOT)], sem)

    def wait_put(slot, sem):
        pltpu.make_async_copy(
            rows_v.at[slot], out_hbm.at[pl.ds(0, SLOT)], sem).wait()

    gather(0, 0, g0)
    gather(1, 1, g1)

    def body(i, _):
        j0 = 2 * i
        wait_gather(0, g0)
        put(j0, 0, o0)
        wait_gather(1, g1)
        put(j0 + 1, 1, o1)

        @pl.when(i + 1 < SPW // 2)
        def _():
            wait_put(0, o0)
            gather(j0 + 2, 0, g0)
            wait_put(1, o1)
            gather(j0 + 3, 1, g1)

        return 0

    lax.fori_loop(0, SPW // 2, body, 0)
    wait_put(0, o0)
    wait_put(1, o1)


def _mask_eos_body(ids_ref, mask_ref, eos_ref):
    ids = ids_ref[...]
    mask_ref[...] = ids == PAD_ID
    eos_ref[...] = (ids == EOS_ID).astype(jnp.float32)


_mask_eos = pl.pallas_call(
    _mask_eos_body,
    out_shape=(
        jax.ShapeDtypeStruct((B, L), jnp.bool_),
        jax.ShapeDtypeStruct((B, L), jnp.float32),
    ),
)


def kernel(lookup_ids, embedding_matrix):
    ids_flat = lookup_ids.reshape(N)
    matrices = _gather_sc(ids_flat, embedding_matrix).reshape(B, L, D)
    mask, eos = _mask_eos(lookup_ids)
    return (matrices, mask, eos, embedding_matrix, lookup_ids)


# speedup vs baseline: 1.4518x; 1.3362x over previous
import functools

import jax
import jax.numpy as jnp
from jax import lax
from jax.experimental import pallas as pl
from jax.experimental.pallas import tpu as pltpu
from jax.experimental.pallas import tpu_sc as plsc

B, L, D, V = 512, 512, 128, 1024
PAD_ID, EOS_ID = 0, 1
N = B * L
NC, NS = 2, 16
NW = NC * NS
SLOT = 128
NBUF = 4
SPW = N // (NW * SLOT)
IPW = N // NW

_mesh = plsc.VectorSubcoreMesh(core_axis_name="c", subcore_axis_name="s")


@functools.partial(
    pl.kernel,
    out_type=jax.ShapeDtypeStruct((N, D), jnp.float32),
    mesh=_mesh,
    scratch_types=[
        pltpu.VMEM((IPW,), jnp.int32),
        pltpu.VMEM((NBUF, SLOT, D), jnp.float32),
        pltpu.VMEM_SHARED((V, D), jnp.float32),
        [pltpu.SemaphoreType.DMA] * NBUF,
        [pltpu.SemaphoreType.DMA] * NBUF,
    ],
)
def _gather_sc(ids_hbm, table_hbm, out_hbm, idx_v, rows_v, tab_sh, gsem, osem):
    wid = lax.axis_index("s") * NC + lax.axis_index("c")
    base_id = wid * IPW

    @pl.when(lax.axis_index("s") == 0)
    def _():
        pltpu.sync_copy(table_hbm, tab_sh)

    pltpu.sync_copy(ids_hbm.at[pl.ds(base_id, IPW)], idx_v)
    plsc.subcore_barrier()

    def gather(j, b):
        pltpu.async_copy(
            tab_sh.at[idx_v.at[pl.ds(j * SLOT, SLOT)]], rows_v.at[b], gsem[b])

    def wait_gather(b):
        pltpu.make_async_copy(
            tab_sh.at[idx_v.at[pl.ds(0, SLOT)]], rows_v.at[b], gsem[b]).wait()

    def put(j, b):
        pltpu.async_copy(
            rows_v.at[b], out_hbm.at[pl.ds(base_id + j * SLOT, SLOT)], osem[b])

    def wait_put(b):
        pltpu.make_async_copy(
            rows_v.at[b], out_hbm.at[pl.ds(0, SLOT)], osem[b]).wait()

    for b in range(NBUF):
        gather(b, b)

    def body(i, _):
        j0 = NBUF * i
        for b in range(NBUF):
            wait_gather(b)
            put(j0 + b, b)

        @pl.when(i + 1 < SPW // NBUF)
        def _():
            for b in range(NBUF):
                wait_put(b)
                gather(j0 + NBUF + b, b)

        return 0

    lax.fori_loop(0, SPW // NBUF, body, 0)
    for b in range(NBUF):
        wait_put(b)


def _mask_eos_body(ids_ref, mask_ref, eos_ref):
    ids = ids_ref[...]
    mask_ref[...] = ids == PAD_ID
    eos_ref[...] = (ids == EOS_ID).astype(jnp.float32)


_mask_eos = pl.pallas_call(
    _mask_eos_body,
    out_shape=(
        jax.ShapeDtypeStruct((B, L), jnp.bool_),
        jax.ShapeDtypeStruct((B, L), jnp.float32),
    ),
)


def kernel(lookup_ids, embedding_matrix):
    ids_flat = lookup_ids.reshape(N)
    matrices = _gather_sc(ids_flat, embedding_matrix).reshape(B, L, D)
    mask, eos = _mask_eos(lookup_ids)
    return (matrices, mask, eos, embedding_matrix, lookup_ids)


# speedup vs baseline: 1.4689x; 1.0118x over previous
import functools

import jax
import jax.numpy as jnp
from jax import lax
from jax.experimental import pallas as pl
from jax.experimental.pallas import tpu as pltpu
from jax.experimental.pallas import tpu_sc as plsc

B, L, D, V = 512, 512, 128, 1024
PAD_ID, EOS_ID = 0, 1
N = B * L
NC, NS = 2, 16
NW = NC * NS
SLOT = 64
NBUF = 8
SPW = N // (NW * SLOT)
IPW = N // NW

_mesh = plsc.VectorSubcoreMesh(core_axis_name="c", subcore_axis_name="s")


@functools.partial(
    pl.kernel,
    out_type=jax.ShapeDtypeStruct((N, D), jnp.float32),
    mesh=_mesh,
    scratch_types=[
        pltpu.VMEM((IPW,), jnp.int32),
        pltpu.VMEM((NBUF, SLOT, D), jnp.float32),
        pltpu.VMEM_SHARED((V, D), jnp.float32),
        [pltpu.SemaphoreType.DMA] * NBUF,
        [pltpu.SemaphoreType.DMA] * NBUF,
    ],
)
def _gather_sc(ids_hbm, table_hbm, out_hbm, idx_v, rows_v, tab_sh, gsem, osem):
    wid = lax.axis_index("s") * NC + lax.axis_index("c")
    base_id = wid * IPW

    @pl.when(lax.axis_index("s") == 0)
    def _():
        pltpu.sync_copy(table_hbm, tab_sh)

    pltpu.sync_copy(ids_hbm.at[pl.ds(base_id, IPW)], idx_v)
    plsc.subcore_barrier()

    def gather(j, b):
        pltpu.async_copy(
            tab_sh.at[idx_v.at[pl.ds(j * SLOT, SLOT)]], rows_v.at[b], gsem[b])

    def wait_gather(b):
        pltpu.make_async_copy(
            tab_sh.at[idx_v.at[pl.ds(0, SLOT)]], rows_v.at[b], gsem[b]).wait()

    def put(j, b):
        pltpu.async_copy(
            rows_v.at[b], out_hbm.at[pl.ds(base_id + j * SLOT, SLOT)], osem[b])

    def wait_put(b):
        pltpu.make_async_copy(
            rows_v.at[b], out_hbm.at[pl.ds(0, SLOT)], osem[b]).wait()

    for b in range(NBUF):
        gather(b, b)

    def body(i, _):
        j0 = NBUF * i
        for b in range(NBUF):
            wait_gather(b)
            put(j0 + b, b)

        @pl.when(i + 1 < SPW // NBUF)
        def _():
            for b in range(NBUF):
                wait_put(b)
                gather(j0 + NBUF + b, b)

        return 0

    lax.fori_loop(0, SPW // NBUF, body, 0)
    for b in range(NBUF):
        wait_put(b)


def _mask_eos_body(ids_ref, mask_ref, eos_ref):
    ids = ids_ref[...]
    mask_ref[...] = ids == PAD_ID
    eos_ref[...] = (ids == EOS_ID).astype(jnp.float32)


_mask_eos = pl.pallas_call(
    _mask_eos_body,
    out_shape=(
        jax.ShapeDtypeStruct((B, L), jnp.bool_),
        jax.ShapeDtypeStruct((B, L), jnp.float32),
    ),
)


def kernel(lookup_ids, embedding_matrix):
    ids_flat = lookup_ids.reshape(N)
    matrices = _gather_sc(ids_flat, embedding_matrix).reshape(B, L, D)
    mask, eos = _mask_eos(lookup_ids)
    return (matrices, mask, eos, embedding_matrix, lookup_ids)
